# baseline (speedup 1.0000x reference)
import functools

import jax
import jax.numpy as jnp
from jax import lax
from jax.experimental import pallas as pl
from jax.experimental.pallas import tpu as pltpu
from jax.experimental.pallas import tpu_sc as plsc

K = 64
N = 32768
B = 128
NS = 16
CHUNK = N // NS
NV = CHUNK // 16
NCV = (NS * K) // 16


def _radix_select(bits_fn, nv, quota, hist_ref, tmp_ref):
    prefix = jnp.int32(0)
    q = jnp.int32(quota)
    zero16 = jnp.zeros((16,), jnp.int32)
    one16 = jnp.ones((16,), jnp.int32)
    lane = jnp.arange(16, dtype=jnp.int32)
    for level in range(4):
        shift = 24 - 8 * level
        nbsel = 4 if level == 0 else 16
        for j in range(nbsel):
            hist_ref[pl.ds(16 * j, 16)] = zero16

        if level == 0:
            @plsc.parallel_loop(0, nv, unroll=8)
            def _build(j):
                v = bits_fn(j)
                binv = (v >> shift) & 255
                plsc.addupdate_scatter(hist_ref, [binv], one16)
        else:
            pfx = prefix
            shift_ = shift

            @plsc.parallel_loop(0, nv, unroll=8)
            def _build(j, _pfx=pfx, _shift=shift_):
                v = bits_fn(j)
                ok = (v >> (_shift + 8)) == _pfx
                binv = (v >> _shift) & 255
                plsc.addupdate_scatter(hist_ref, [binv], one16, mask=ok)

        @plsc.parallel_loop(0, nbsel, unroll=4,
                            carry=(jnp.int32(0), zero16, zero16))
        def _sel(i, carry, _q=q, _nb=nbsel):
            above, b_accv, s_accv = carry
            i2 = _nb - 1 - i
            v = hist_ref[pl.ds(16 * i2, 16)]
            cs = plsc.cumsum(v)
            total = cs[15]
            sfx = (above + total) - cs
            cond = ((sfx < _q) & (sfx + v >= _q)).astype(jnp.int32)
            b_accv = b_accv + (i2 * 16 + lane) * cond
            s_accv = s_accv + sfx * cond
            return above + total, b_accv, s_accv

        _, b_accv, s_accv = _sel
        bstar = jnp.sum(b_accv)
        q = q - jnp.sum(s_accv)
        prefix = (prefix << 8) | bstar
    return prefix, q


def _compact(val_fn, idx_fn, nv, t_bits, need, outv_ref, outi_ref, tmp_ref):
    def one(j, eq_seen, sel_seen):
        xv = val_fn(j)
        bv = lax.bitcast_convert_type(xv, jnp.int32)
        gt = bv > t_bits
        eq = bv == t_bits
        eqi = eq.astype(jnp.int32)
        ecs = plsc.cumsum(eqi)
        eq_excl = ecs - eqi
        sel = gt | (eq & ((eq_seen + eq_excl) < need))
        seln = sel.astype(jnp.int32)
        scs = plsc.cumsum(seln)
        sel_excl = scs - seln
        slot = sel_seen + sel_excl
        plsc.store_scatter(outv_ref, [slot], xv, mask=sel)
        plsc.store_scatter(outi_ref, [slot], idx_fn(j), mask=sel)
        return eq_seen + ecs[15], sel_seen + scs[15]

    @plsc.parallel_loop(0, nv, unroll=8, carry=(jnp.int32(0), jnp.int32(0)))
    def _run(j, carry):
        eq_seen, sel_seen = carry
        return one(j, eq_seen, sel_seen)


def _sc_body(x_hbm, outv_hbm, outi_hbm, xb, hist, candv_l, candi_l,
             cand_sh_v, cand_sh_i, candv, candi, selv, seli, tmp):
    s = lax.axis_index("s")
    lane = jnp.arange(16, dtype=jnp.int32)

    pltpu.sync_copy(x_hbm.at[pl.ds(s * CHUNK, CHUNK)], xb)

    def my_bits(j):
        return lax.bitcast_convert_type(xb[pl.ds(16 * j, 16)], jnp.int32)

    t_loc, need_loc = _radix_select(my_bits, NV, K, hist, tmp)
    base = s * CHUNK
    _compact(lambda j: xb[pl.ds(16 * j, 16)],
             lambda j: base + j * 16 + lane,
             NV, t_loc, need_loc, candv_l, candi_l, tmp)

    pltpu.sync_copy(candv_l, cand_sh_v.at[pl.ds(s * K, K)])
    pltpu.sync_copy(candi_l, cand_sh_i.at[pl.ds(s * K, K)])
    plsc.subcore_barrier()
    pltpu.sync_copy(cand_sh_v, candv)
    pltpu.sync_copy(cand_sh_i, candi)

    def cand_bits(j):
        return lax.bitcast_convert_type(candv[pl.ds(16 * j, 16)], jnp.int32)

    t_g, need_g = _radix_select(cand_bits, NCV, K, hist, tmp)
    _compact(lambda j: candv[pl.ds(16 * j, 16)],
             lambda j: candi[pl.ds(16 * j, 16)],
             NCV, t_g, need_g, selv, seli, tmp)

    @pl.when(s == 0)
    def _out():
        pltpu.sync_copy(selv, outv_hbm)
        pltpu.sync_copy(seli, outi_hbm)


@functools.partial(
    pl.kernel,
    out_type=(jax.ShapeDtypeStruct((K,), jnp.float32),
              jax.ShapeDtypeStruct((K,), jnp.int32)),
    mesh=plsc.VectorSubcoreMesh(core_axis_name="c", subcore_axis_name="s",
                                num_cores=1),
    compiler_params=pltpu.CompilerParams(needs_layout_passes=False),
    scratch_types=[
        pltpu.VMEM((CHUNK,), jnp.float32),
        pltpu.VMEM((256,), jnp.int32),
        pltpu.VMEM((K,), jnp.float32),
        pltpu.VMEM((K,), jnp.int32),
        pltpu.VMEM_SHARED((NS * K,), jnp.float32),
        pltpu.VMEM_SHARED((NS * K,), jnp.int32),
        pltpu.VMEM((NS * K,), jnp.float32),
        pltpu.VMEM((NS * K,), jnp.int32),
        pltpu.VMEM((K,), jnp.float32),
        pltpu.VMEM((K,), jnp.int32),
        pltpu.VMEM((16,), jnp.int32),
    ],
)
def _sc_select(x_hbm, outv_hbm, outi_hbm, *rest):
    _sc_body(x_hbm, outv_hbm, outi_hbm, *rest)


GW = 32
GSTEPS = K // GW


def _gather_kernel(seli_ref, selv_ref, *refs):
    oracle_refs = refs[:GW]
    out_ref = refs[GW]
    acc_ref = refs[GW + 1]
    i = pl.program_id(0)

    @pl.when(i == 0)
    def _init():
        acc_ref[...] = jnp.zeros_like(acc_ref)

    lanes = lax.broadcasted_iota(jnp.int32, (B, 128), 1)
    ii = lax.broadcasted_iota(jnp.int32, (1, K), 1)
    acc = acc_ref[...]
    selvv = selv_ref[...]
    for u in range(GW):
        col = seli_ref[i * GW + u] % 128
        w = jnp.sum(jnp.where(ii == i * GW + u, selvv, 0.0))
        acc += jnp.where(lanes == col, oracle_refs[u][...], 0.0) * w
    acc_ref[...] = acc

    @pl.when(i == GSTEPS - 1)
    def _fin():
        sample = jnp.sum(acc_ref[...], axis=1, keepdims=True)
        out_ref[...] = jnp.sum(jnp.log(sample + 1e-10), keepdims=True) / B


def kernel(concat_output, oracle_prob, k):
    selv, seli = _sc_select(concat_output)
    def _mk_spec(u):
        return pl.BlockSpec((B, 128), lambda i, s, _u=u: (0, s[i * GW + _u] // 128))

    out = pl.pallas_call(
        _gather_kernel,
        grid_spec=pltpu.PrefetchScalarGridSpec(
            num_scalar_prefetch=1,
            grid=(GSTEPS,),
            in_specs=[pl.BlockSpec((1, K), lambda i, s: (0, 0))]
                     + [_mk_spec(u) for u in range(GW)],
            out_specs=pl.BlockSpec((1, 1), lambda i, s: (0, 0)),
            scratch_shapes=[pltpu.VMEM((B, 128), jnp.float32)],
        ),
        out_shape=jax.ShapeDtypeStruct((1, 1), jnp.float32),
    )(seli, selv.reshape(1, K), *([oracle_prob] * GW))
    return out[0, 0]


# speedup vs baseline: 1.0286x; 1.0286x over previous
import functools

import jax
import jax.numpy as jnp
from jax import lax
from jax.experimental import pallas as pl
from jax.experimental.pallas import tpu as pltpu
from jax.experimental.pallas import tpu_sc as plsc

K = 64
N = 32768
B = 128
NS = 16
CHUNK = N // NS
NV = CHUNK // 16
NCV = (NS * K) // 16


def _radix_select(bits_fn, nv, quota, hist_ref, tmp_ref):
    prefix = jnp.int32(0)
    q = jnp.int32(quota)
    zero16 = jnp.zeros((16,), jnp.int32)
    one16 = jnp.ones((16,), jnp.int32)
    lane = jnp.arange(16, dtype=jnp.int32)
    for level in range(4):
        shift = 24 - 8 * level
        nbsel = 4 if level == 0 else 16
        for j in range(nbsel):
            hist_ref[pl.ds(16 * j, 16)] = zero16

        if level == 0:
            @plsc.parallel_loop(0, nv, unroll=4)
            def _build(j):
                v = bits_fn(j)
                binv = (v >> shift) & 255
                plsc.addupdate_scatter(hist_ref, [binv], one16)
        else:
            pfx = prefix
            shift_ = shift

            @plsc.parallel_loop(0, nv, unroll=4)
            def _build(j, _pfx=pfx, _shift=shift_):
                v = bits_fn(j)
                ok = (v >> (_shift + 8)) == _pfx
                binv = (v >> _shift) & 255
                plsc.addupdate_scatter(hist_ref, [binv], one16, mask=ok)

        @plsc.parallel_loop(0, nbsel, unroll=4,
                            carry=(jnp.int32(0), zero16, zero16))
        def _sel(i, carry, _q=q, _nb=nbsel):
            above, b_accv, s_accv = carry
            i2 = _nb - 1 - i
            v = hist_ref[pl.ds(16 * i2, 16)]
            cs = plsc.cumsum(v)
            total = cs[15]
            sfx = (above + total) - cs
            cond = ((sfx < _q) & (sfx + v >= _q)).astype(jnp.int32)
            b_accv = b_accv + (i2 * 16 + lane) * cond
            s_accv = s_accv + sfx * cond
            return above + total, b_accv, s_accv

        _, b_accv, s_accv = _sel
        bstar = jnp.sum(b_accv)
        q = q - jnp.sum(s_accv)
        prefix = (prefix << 8) | bstar
    return prefix, q


def _compact(val_fn, idx_fn, nv, t_bits, need, outv_ref, outi_ref, tmp_ref):
    def one(j, eq_seen, sel_seen):
        xv = val_fn(j)
        bv = lax.bitcast_convert_type(xv, jnp.int32)
        gt = bv > t_bits
        eq = bv == t_bits
        eqi = eq.astype(jnp.int32)
        ecs = plsc.cumsum(eqi)
        eq_excl = ecs - eqi
        sel = gt | (eq & ((eq_seen + eq_excl) < need))
        seln = sel.astype(jnp.int32)
        scs = plsc.cumsum(seln)
        sel_excl = scs - seln
        slot = sel_seen + sel_excl
        plsc.store_scatter(outv_ref, [slot], xv, mask=sel)
        plsc.store_scatter(outi_ref, [slot], idx_fn(j), mask=sel)
        return eq_seen + ecs[15], sel_seen + scs[15]

    @plsc.parallel_loop(0, nv, unroll=4, carry=(jnp.int32(0), jnp.int32(0)))
    def _run(j, carry):
        eq_seen, sel_seen = carry
        return one(j, eq_seen, sel_seen)


def _sc_body(x_hbm, outv_hbm, outi_hbm, xb, hist, candv_l, candi_l,
             cand_sh_v, cand_sh_i, candv, candi, selv, seli, tmp):
    s = lax.axis_index("s")
    lane = jnp.arange(16, dtype=jnp.int32)

    pltpu.sync_copy(x_hbm.at[pl.ds(s * CHUNK, CHUNK)], xb)

    def my_bits(j):
        return lax.bitcast_convert_type(xb[pl.ds(16 * j, 16)], jnp.int32)

    t_loc, need_loc = _radix_select(my_bits, NV, K, hist, tmp)
    base = s * CHUNK
    _compact(lambda j: xb[pl.ds(16 * j, 16)],
             lambda j: base + j * 16 + lane,
             NV, t_loc, need_loc, candv_l, candi_l, tmp)

    pltpu.sync_copy(candv_l, cand_sh_v.at[pl.ds(s * K, K)])
    pltpu.sync_copy(candi_l, cand_sh_i.at[pl.ds(s * K, K)])
    plsc.subcore_barrier()
    pltpu.sync_copy(cand_sh_v, candv)
    pltpu.sync_copy(cand_sh_i, candi)

    def cand_bits(j):
        return lax.bitcast_convert_type(candv[pl.ds(16 * j, 16)], jnp.int32)

    t_g, need_g = _radix_select(cand_bits, NCV, K, hist, tmp)
    _compact(lambda j: candv[pl.ds(16 * j, 16)],
             lambda j: candi[pl.ds(16 * j, 16)],
             NCV, t_g, need_g, selv, seli, tmp)

    @pl.when(s == 0)
    def _out():
        pltpu.sync_copy(selv, outv_hbm)
        pltpu.sync_copy(seli, outi_hbm)


@functools.partial(
    pl.kernel,
    out_type=(jax.ShapeDtypeStruct((K,), jnp.float32),
              jax.ShapeDtypeStruct((K,), jnp.int32)),
    mesh=plsc.VectorSubcoreMesh(core_axis_name="c", subcore_axis_name="s",
                                num_cores=1),
    compiler_params=pltpu.CompilerParams(needs_layout_passes=False),
    scratch_types=[
        pltpu.VMEM((CHUNK,), jnp.float32),
        pltpu.VMEM((256,), jnp.int32),
        pltpu.VMEM((K,), jnp.float32),
        pltpu.VMEM((K,), jnp.int32),
        pltpu.VMEM_SHARED((NS * K,), jnp.float32),
        pltpu.VMEM_SHARED((NS * K,), jnp.int32),
        pltpu.VMEM((NS * K,), jnp.float32),
        pltpu.VMEM((NS * K,), jnp.int32),
        pltpu.VMEM((K,), jnp.float32),
        pltpu.VMEM((K,), jnp.int32),
        pltpu.VMEM((16,), jnp.int32),
    ],
)
def _sc_select(x_hbm, outv_hbm, outi_hbm, *rest):
    _sc_body(x_hbm, outv_hbm, outi_hbm, *rest)


GW = 32
GSTEPS = K // GW


def _gather_kernel(seli_ref, selv_ref, *refs):
    oracle_refs = refs[:GW]
    out_ref = refs[GW]
    acc_ref = refs[GW + 1]
    i = pl.program_id(0)

    @pl.when(i == 0)
    def _init():
        acc_ref[...] = jnp.zeros_like(acc_ref)

    lanes = lax.broadcasted_iota(jnp.int32, (B, 128), 1)
    ii = lax.broadcasted_iota(jnp.int32, (1, K), 1)
    acc = acc_ref[...]
    selvv = selv_ref[...]
    for u in range(GW):
        col = seli_ref[i * GW + u] % 128
        w = jnp.sum(jnp.where(ii == i * GW + u, selvv, 0.0))
        acc += jnp.where(lanes == col, oracle_refs[u][...], 0.0) * w
    acc_ref[...] = acc

    @pl.when(i == GSTEPS - 1)
    def _fin():
        sample = jnp.sum(acc_ref[...], axis=1, keepdims=True)
        out_ref[...] = jnp.sum(jnp.log(sample + 1e-10), keepdims=True) / B


def kernel(concat_output, oracle_prob, k):
    selv, seli = _sc_select(concat_output)
    def _mk_spec(u):
        return pl.BlockSpec((B, 128), lambda i, s, _u=u: (0, s[i * GW + _u] // 128))

    out = pl.pallas_call(
        _gather_kernel,
        grid_spec=pltpu.PrefetchScalarGridSpec(
            num_scalar_prefetch=1,
            grid=(GSTEPS,),
            in_specs=[pl.BlockSpec((1, K), lambda i, s: (0, 0))]
                     + [_mk_spec(u) for u in range(GW)],
            out_specs=pl.BlockSpec((1, 1), lambda i, s: (0, 0)),
            scratch_shapes=[pltpu.VMEM((B, 128), jnp.float32)],
        ),
        out_shape=jax.ShapeDtypeStruct((1, 1), jnp.float32),
    )(seli, selv.reshape(1, K), *([oracle_prob] * GW))
    return out[0, 0]
